# Initial kernel scaffold; baseline (speedup 1.0000x reference)
import jax
import jax.numpy as jnp
from jax.experimental import pallas as pl


def kernel(box_cls, box_center, border_cls, border_delta, bd_based_box, image_sizes):
    raise NotImplementedError("write your pallas kernel here")


# speedup vs baseline: 6.1965x; 6.1965x over previous
import jax
import jax.numpy as jnp
from jax.experimental import pallas as pl

_PRE_NMS_THRESH = 0.05
_PRE_NMS_TOP_N = 1000
_NMS_THRESH = 0.6
_POST_TOP_N = 100
_NUM_CLASSES = 80
_N = 1024
_CHUNK = 1024
_HWPAD = 15360


def _score_kernel(cls_ref, ctr_ref, bcls_ref, out_ref):
    cls = jax.nn.sigmoid(cls_ref[...])
    ctr = jax.nn.sigmoid(ctr_ref[...])
    bcls = jax.nn.sigmoid(bcls_ref[...])
    pred = jnp.sqrt(cls * ctr)
    out_ref[...] = jnp.where(pred > _PRE_NMS_THRESH, pred * bcls, -1.0)


def _nms_kernel(vals_ref, bidx_ref, cidx_ref, br_ref, hw_ref, out_ref):
    vals = vals_ref[...]
    bidx = bidx_ref[...]
    cidx = cidx_ref[...]
    jidx = jax.lax.broadcasted_iota(jnp.int32, (1, _N), 1)

    gath = jnp.zeros((8, _N), jnp.float32)
    for c in range(_HWPAD // _CHUNK):
        lo = c * _CHUNK
        sel = (bidx == (jax.lax.broadcasted_iota(jnp.int32, (_CHUNK, 1), 0) + lo))
        onehot = sel.astype(jnp.float32)
        gath = gath + jax.lax.dot_general(
            br_ref[lo:lo + _CHUNK, :], onehot,
            (((0,), (0,)), ((), ())),
            preferred_element_type=jnp.float32)

    x1b, y1b = gath[0:1, :], gath[1:2, :]
    x2b, y2b = gath[2:3, :], gath[3:4, :]
    rx1, ry1 = gath[4:5, :], gath[5:6, :]
    rx2, ry2 = gath[6:7, :], gath[7:8, :]
    bw = x2b - x1b
    bh = y2b - y1b
    x1 = x1b + rx1 * 0.1 * bw
    y1 = y1b + ry1 * 0.1 * bh
    x2 = x2b + rx2 * 0.2 * bw
    y2 = y2b + ry2 * 0.2 * bh

    valid = vals > 0.0
    s = jnp.where(valid, jnp.sqrt(jnp.maximum(vals, 0.0)), 0.0)

    in_range = jidx < _PRE_NMS_TOP_N
    neg = jnp.float32(-1e30)
    mc = jnp.maximum(
        jnp.maximum(jnp.max(jnp.where(in_range, x1, neg)),
                    jnp.max(jnp.where(in_range, y1, neg))),
        jnp.maximum(jnp.max(jnp.where(in_range, x2, neg)),
                    jnp.max(jnp.where(in_range, y2, neg)))) + 1.0
    off = cidx * mc
    nx1, ny1, nx2, ny2 = x1 + off, y1 + off, x2 + off, y2 + off
    areas = jnp.maximum(nx2 - nx1, 0.0) * jnp.maximum(ny2 - ny1, 0.0)

    def body(j, state):
        supp, keep = state
        mj = (jidx == j).astype(jnp.float32)
        sj = jnp.sum(s * mj)
        suppj = jnp.sum(supp * mj)
        take = jnp.where((suppj < 0.5) & (sj > 0.0), 1.0, 0.0)
        keep = keep + take * mj
        xx1 = jnp.maximum(jnp.sum(nx1 * mj), nx1)
        yy1 = jnp.maximum(jnp.sum(ny1 * mj), ny1)
        xx2 = jnp.minimum(jnp.sum(nx2 * mj), nx2)
        yy2 = jnp.minimum(jnp.sum(ny2 * mj), ny2)
        inter = jnp.maximum(xx2 - xx1, 0.0) * jnp.maximum(yy2 - yy1, 0.0)
        iou = inter / (jnp.sum(areas * mj) + areas - inter + 1e-9)
        supp = jnp.maximum(supp, jnp.where(iou > _NMS_THRESH, take, 0.0))
        return supp, keep

    supp0 = jnp.zeros((1, _N), jnp.float32)
    keep0 = jnp.zeros((1, _N), jnp.float32)
    _, keep = jax.lax.fori_loop(0, _PRE_NMS_TOP_N, body, (supp0, keep0))

    ut = (jax.lax.broadcasted_iota(jnp.int32, (_N, 1), 0)
          <= jax.lax.broadcasted_iota(jnp.int32, (1, _N), 1)).astype(jnp.float32)
    kr = jnp.dot(keep, ut, preferred_element_type=jnp.float32)
    sel1 = keep * (kr <= float(_POST_TOP_N)).astype(jnp.float32)

    hgt = jnp.sum(hw_ref[0:1, 0:1])
    wid = jnp.sum(hw_ref[0:1, 1:2])
    cx1 = jnp.clip(x1, 0.0, wid)
    cy1 = jnp.clip(y1, 0.0, hgt)
    cx2 = jnp.clip(x2, 0.0, wid)
    cy2 = jnp.clip(y2, 0.0, hgt)
    ok = ((cx2 - cx1) >= 0.0) & ((cy2 - cy1) >= 0.0)
    final = sel1 * ok.astype(jnp.float32)

    fc = jnp.dot(final, ut, preferred_element_type=jnp.float32)
    rows = fc - 1.0
    rcol = jax.lax.broadcasted_iota(jnp.int32, (128, 1), 0).astype(jnp.float32)
    m = (rcol == rows).astype(jnp.float32) * final
    lab = cidx + 1.0
    zeros = jnp.zeros((1, _N), jnp.float32)
    v = jnp.concatenate([cx1, cy1, cx2, cy2, s, lab, zeros, zeros], axis=0)
    out_ref[...] = jax.lax.dot_general(
        m, v, (((1,), (1,)), ((), ())), preferred_element_type=jnp.float32)


def kernel(box_cls, box_center, border_cls, border_delta, bd_based_box, image_sizes):
    K = _NUM_CLASSES
    hw = box_cls.shape[2] * box_cls.shape[3]

    cls2 = box_cls[0].reshape(K, hw)
    ctr2 = box_center[0].reshape(1, hw)
    bcls2 = border_cls[0].reshape(K, hw)

    masked = pl.pallas_call(
        _score_kernel,
        out_shape=jax.ShapeDtypeStruct((K, hw), jnp.float32),
    )(cls2, ctr2, bcls2)

    flat = masked.T.reshape(-1)
    vals, idxs = jax.lax.top_k(flat, _PRE_NMS_TOP_N)
    box_idx = idxs // K
    cls_idx = idxs % K

    pad = _N - _PRE_NMS_TOP_N
    vals_p = jnp.concatenate([vals, jnp.full((pad,), -1.0, jnp.float32)]).reshape(1, _N)
    bidx_p = jnp.concatenate([box_idx, jnp.zeros((pad,), box_idx.dtype)]).reshape(1, _N).astype(jnp.int32)
    cidx_p = jnp.concatenate([cls_idx, jnp.zeros((pad,), cls_idx.dtype)]).reshape(1, _N).astype(jnp.float32)

    base = bd_based_box[0]
    reg = border_delta[0].reshape(4, hw).T
    br = jnp.concatenate([base, reg], axis=1)
    br = jnp.pad(br, ((0, _HWPAD - hw), (0, 0)))
    hw_f = image_sizes.astype(jnp.float32)

    out = pl.pallas_call(
        _nms_kernel,
        out_shape=jax.ShapeDtypeStruct((128, 8), jnp.float32),
    )(vals_p, bidx_p, cidx_p, br, hw_f)
    return out[:_POST_TOP_N, :6]
